# Initial kernel scaffold; baseline (speedup 1.0000x reference)
import jax
import jax.numpy as jnp
from jax.experimental import pallas as pl


def kernel(x, edge_index, edge_attr, W1, b1, gamma, beta, W2, b2, eps_p):
    raise NotImplementedError("write your pallas kernel here")


# speedup vs baseline: 4.8898x; 4.8898x over previous
import functools

import jax
import jax.numpy as jnp
from jax import lax
from jax.experimental import pallas as pl
from jax.experimental.pallas import tpu as pltpu
from jax.experimental.pallas import tpu_sc as plsc

D = 128
N = 10000
E = 320000

NC = 2
NS = 16
L = 16
NW = NC * NS
EPW = E // NW
C = 80
CH = EPW // C
SS = 5
SCH = CH // SS
NACC = 10240
NPW = NACC // NS
BN = 400
NBLK = N // BN


def _edge_phase(x, row3, col3, edge_attr):
    mesh = plsc.VectorSubcoreMesh(core_axis_name="c", subcore_axis_name="s")

    @functools.partial(
        pl.kernel,
        mesh=mesh,
        out_type=jax.ShapeDtypeStruct((2 * NACC, D), jnp.float32),
        scratch_types=[
            pltpu.VMEM_SHARED((NACC, D), jnp.float32),
            pltpu.VMEM((SCH, C), jnp.int32),
            pltpu.VMEM((SCH, C), jnp.int32),
            pltpu.VMEM((C, D), jnp.float32),
            pltpu.VMEM((C, D), jnp.float32),
            pltpu.SemaphoreType.DMA,
            pltpu.SemaphoreType.DMA,
        ],
    )
    def k(x_hbm, row_hbm, col_hbm, ea_hbm, out_hbm,
          acc_sh, row_v, col_v, xg_v, ea_v, sem0, sem1):
        c = lax.axis_index("c")
        s = lax.axis_index("s")
        wid = c * NS + s

        def zrow(i, _):
            for j in range(D // L):
                xg_v[i, pl.ds(j * L, L)] = jnp.zeros((L,), jnp.float32)
            return 0

        lax.fori_loop(0, C, zrow, 0)
        for r in range(NPW // C):
            pltpu.sync_copy(xg_v, acc_sh.at[pl.ds(s * NPW + r * C, C)])
        plsc.subcore_barrier()

        def superstep(ss, _):
            pltpu.sync_copy(row_hbm.at[wid, ss], row_v)
            pltpu.sync_copy(col_hbm.at[wid, ss], col_v)

            def chunk(kk, _):
                ebase = wid * EPW + (ss * SCH + kk) * C
                cp_ea = pltpu.async_copy(ea_hbm.at[pl.ds(ebase, C)], ea_v,
                                         sem0)
                cp_g = pltpu.async_copy(x_hbm.at[col_v.at[kk]], xg_v, sem1)
                cp_ea.wait()
                cp_g.wait()

                def rowfn(i, _):
                    for j in range(D // L):
                        sl = pl.ds(j * L, L)
                        xg_v[i, sl] = jnp.maximum(xg_v[i, sl] + ea_v[i, sl],
                                                  0.0)
                    return 0

                lax.fori_loop(0, C, rowfn, 0)
                pltpu.sync_copy(xg_v, acc_sh.at[row_v.at[kk]], add=True)
                return 0

            lax.fori_loop(0, SCH, chunk, 0)
            return 0

        lax.fori_loop(0, SS, superstep, 0)
        plsc.subcore_barrier()
        pltpu.sync_copy(acc_sh.at[pl.ds(s * NPW, NPW)],
                        out_hbm.at[pl.ds(c * NACC + s * NPW, NPW)])

    return k(x, row3, col3, edge_attr)


def _mlp_body(scale_ref, x_ref, p0_ref, p1_ref, W1_ref, b1_ref, g_ref,
              be_ref, W2_ref, b2_ref, o_ref):
    acc = p0_ref[...] + p1_ref[...]
    h = scale_ref[0, 0] * x_ref[...] + jnp.maximum(acc, 0.0)
    h1 = jnp.dot(h, W1_ref[...], preferred_element_type=jnp.float32)
    h1 = h1 + b1_ref[...]
    mu = jnp.mean(h1, axis=-1, keepdims=True)
    d = h1 - mu
    var = jnp.mean(d * d, axis=-1, keepdims=True)
    h1n = d * lax.rsqrt(var + 1e-5) * g_ref[...] + be_ref[...]
    o = jnp.dot(jnp.maximum(h1n, 0.0), W2_ref[...],
                preferred_element_type=jnp.float32)
    o_ref[...] = o + b2_ref[...]


def _node_phase(x, p0, p1, W1, b1, gamma, beta, W2, b2, eps_p):
    scale = (1.0 + eps_p).reshape(1, 1)
    return pl.pallas_call(
        _mlp_body,
        grid=(NBLK,),
        in_specs=[
            pl.BlockSpec((1, 1), lambda i: (0, 0)),
            pl.BlockSpec((BN, D), lambda i: (i, 0)),
            pl.BlockSpec((BN, D), lambda i: (i, 0)),
            pl.BlockSpec((BN, D), lambda i: (i, 0)),
            pl.BlockSpec((D, 2 * D), lambda i: (0, 0)),
            pl.BlockSpec((1, 2 * D), lambda i: (0, 0)),
            pl.BlockSpec((1, 2 * D), lambda i: (0, 0)),
            pl.BlockSpec((1, 2 * D), lambda i: (0, 0)),
            pl.BlockSpec((2 * D, D), lambda i: (0, 0)),
            pl.BlockSpec((1, D), lambda i: (0, 0)),
        ],
        out_specs=pl.BlockSpec((BN, D), lambda i: (i, 0)),
        out_shape=jax.ShapeDtypeStruct((N, D), jnp.float32),
    )(scale, x, p0, p1, W1, b1.reshape(1, -1),
      gamma.reshape(1, -1), beta.reshape(1, -1), W2, b2.reshape(1, -1))


def kernel(x, edge_index, edge_attr, W1, b1, gamma, beta, W2, b2, eps_p):
    row3 = edge_index[0].reshape(NW, SS, SCH, C)
    col3 = edge_index[1].reshape(NW, SS, SCH, C)
    partials = _edge_phase(x, row3, col3, edge_attr)
    p0 = lax.slice(partials, (0, 0), (N, D))
    p1 = lax.slice(partials, (NACC, 0), (NACC + N, D))
    return _node_phase(x, p0, p1, W1, b1, gamma, beta, W2, b2, eps_p)


# speedup vs baseline: 7.8864x; 1.6128x over previous
import functools

import jax
import jax.numpy as jnp
from jax import lax
from jax.experimental import pallas as pl
from jax.experimental.pallas import tpu as pltpu
from jax.experimental.pallas import tpu_sc as plsc

D = 128
N = 10000
E = 320000

NC = 2
NS = 16
L = 16
NW = NC * NS
EPW = E // NW
C = 80
CH = EPW // C
NACC = 10240
NPW = NACC // NS
BN = 400
NBLK = N // BN


def _edge_phase(x, row3, col3, edge_attr):
    mesh = plsc.VectorSubcoreMesh(core_axis_name="c", subcore_axis_name="s")

    @functools.partial(
        pl.kernel,
        mesh=mesh,
        out_type=jax.ShapeDtypeStruct((2 * NACC, D), jnp.float32),
        scratch_types=[
            pltpu.VMEM_SHARED((NACC, D), jnp.float32),
            pltpu.VMEM((C,), jnp.int32),
            pltpu.VMEM((C,), jnp.int32),
            pltpu.VMEM((C,), jnp.int32),
            pltpu.VMEM((C,), jnp.int32),
            pltpu.VMEM((C, D), jnp.float32),
            pltpu.VMEM((C, D), jnp.float32),
            pltpu.VMEM((C, D), jnp.float32),
            pltpu.VMEM((C, D), jnp.float32),
        ] + [pltpu.SemaphoreType.DMA] * 10,
    )
    def k(x_hbm, row_hbm, col_hbm, ea_hbm, out_hbm, acc_sh,
          rb0, rb1, cb0, cb1, xg0, xg1, ea0, ea1,
          sr0, sr1, sc0, sc1, sg0, sg1, se0, se1, ss0, ss1):
        c = lax.axis_index("c")
        s = lax.axis_index("s")
        wid = c * NS + s
        ebase = wid * EPW
        rb, cb, xg, ea = (rb0, rb1), (cb0, cb1), (xg0, xg1), (ea0, ea1)
        sr, sc, sg, se, ssc = (sr0, sr1), (sc0, sc1), (sg0, sg1), \
            (se0, se1), (ss0, ss1)

        def zrow(i, _):
            for j in range(D // L):
                xg0[i, pl.ds(j * L, L)] = jnp.zeros((L,), jnp.float32)
            return 0

        lax.fori_loop(0, C, zrow, 0)
        for r in range(NPW // C):
            pltpu.sync_copy(xg0, acc_sh.at[pl.ds(s * NPW + r * C, C)])
        plsc.subcore_barrier()

        def gather_cp(kk, b):
            return pltpu.make_async_copy(x_hbm.at[cb[b]], xg[b], sg[b])

        def ea_cp(kk, b):
            return pltpu.make_async_copy(
                ea_hbm.at[pl.ds(ebase + kk * C, C)], ea[b], se[b])

        def row_cp(kk, b):
            return pltpu.make_async_copy(
                row_hbm.at[pl.ds(ebase + kk * C, C)], rb[b], sr[b])

        def col_cp(kk, b):
            return pltpu.make_async_copy(
                col_hbm.at[pl.ds(ebase + kk * C, C)], cb[b], sc[b])

        def scat_cp(b):
            return pltpu.make_async_copy(xg[b], acc_sh.at[rb[b]], ssc[b])

        def step(kk, b, has_next):
            nb = 1 - b
            if has_next:
                col_cp(kk + 1, nb).wait()

                @pl.when(kk >= 1)
                def _():
                    scat_cp(nb).wait()

                gather_cp(kk + 1, nb).start()
                ea_cp(kk + 1, nb).start()
                row_cp(kk + 1, nb).start()
            else:
                scat_cp(nb).wait()
            gather_cp(kk, b).wait()
            ea_cp(kk, b).wait()
            if has_next:
                @pl.when(kk <= CH - 3)
                def _():
                    col_cp(kk + 2, b).start()

            def rowfn(i, _):
                for u in range(2):
                    for j in range(D // L):
                        sl = pl.ds(j * L, L)
                        xg[b][2 * i + u, sl] = jnp.maximum(
                            xg[b][2 * i + u, sl] + ea[b][2 * i + u, sl], 0.0)
                return 0

            lax.fori_loop(0, C // 2, rowfn, 0)
            row_cp(kk, b).wait()
            scat_cp(b).start(add=True)

        pltpu.sync_copy(col_hbm.at[pl.ds(ebase, C)], cb0)
        row_cp(0, 0).start()
        gather_cp(0, 0).start()
        ea_cp(0, 0).start()
        col_cp(1, 1).start()

        def pair(p, _):
            step(2 * p, 0, True)
            step(2 * p + 1, 1, True)
            return 0

        lax.fori_loop(0, (CH - 1) // 2, pair, 0)
        step(CH - 1, 0, False)
        scat_cp(0).wait()
        plsc.subcore_barrier()
        pltpu.sync_copy(acc_sh.at[pl.ds(s * NPW, NPW)],
                        out_hbm.at[pl.ds(c * NACC + s * NPW, NPW)])

    return k(x, row3, col3, edge_attr)


def _mlp_body(scale_ref, x_ref, p0_ref, p1_ref, W1_ref, b1_ref, g_ref,
              be_ref, W2_ref, b2_ref, o_ref):
    acc = p0_ref[...] + p1_ref[...]
    h = scale_ref[0, 0] * x_ref[...] + jnp.maximum(acc, 0.0)
    h1 = jnp.dot(h, W1_ref[...], preferred_element_type=jnp.float32)
    h1 = h1 + b1_ref[...]
    mu = jnp.mean(h1, axis=-1, keepdims=True)
    d = h1 - mu
    var = jnp.mean(d * d, axis=-1, keepdims=True)
    h1n = d * lax.rsqrt(var + 1e-5) * g_ref[...] + be_ref[...]
    o = jnp.dot(jnp.maximum(h1n, 0.0), W2_ref[...],
                preferred_element_type=jnp.float32)
    o_ref[...] = o + b2_ref[...]


def _node_phase(x, p0, p1, W1, b1, gamma, beta, W2, b2, eps_p):
    scale = (1.0 + eps_p).reshape(1, 1)
    return pl.pallas_call(
        _mlp_body,
        grid=(NBLK,),
        in_specs=[
            pl.BlockSpec((1, 1), lambda i: (0, 0)),
            pl.BlockSpec((BN, D), lambda i: (i, 0)),
            pl.BlockSpec((BN, D), lambda i: (i, 0)),
            pl.BlockSpec((BN, D), lambda i: (i, 0)),
            pl.BlockSpec((D, 2 * D), lambda i: (0, 0)),
            pl.BlockSpec((1, 2 * D), lambda i: (0, 0)),
            pl.BlockSpec((1, 2 * D), lambda i: (0, 0)),
            pl.BlockSpec((1, 2 * D), lambda i: (0, 0)),
            pl.BlockSpec((2 * D, D), lambda i: (0, 0)),
            pl.BlockSpec((1, D), lambda i: (0, 0)),
        ],
        out_specs=pl.BlockSpec((BN, D), lambda i: (i, 0)),
        out_shape=jax.ShapeDtypeStruct((N, D), jnp.float32),
    )(scale, x, p0, p1, W1, b1.reshape(1, -1),
      gamma.reshape(1, -1), beta.reshape(1, -1), W2, b2.reshape(1, -1))


def kernel(x, edge_index, edge_attr, W1, b1, gamma, beta, W2, b2, eps_p):
    partials = _edge_phase(x, edge_index[0], edge_index[1], edge_attr)
    p0 = lax.slice(partials, (0, 0), (N, D))
    p1 = lax.slice(partials, (NACC, 0), (NACC + N, D))
    return _node_phase(x, p0, p1, W1, b1, gamma, beta, W2, b2, eps_p)
